# Initial kernel scaffold; baseline (speedup 1.0000x reference)
import jax
import jax.numpy as jnp
from jax.experimental import pallas as pl


def kernel(p1, p2, p3, W, b, C_bank):
    raise NotImplementedError("write your pallas kernel here")


# speedup vs baseline: 56.1606x; 56.1606x over previous
import numpy as np
import jax
import jax.numpy as jnp
from jax import lax
from jax.experimental import pallas as pl

B = 8
H = 56
S = H * H
D = 224
C1, C2, C3 = 256, 512, 1024
S2, S3 = 28 * 28, 14 * 14
KNN = 3
R = 784
BIG = 3.0e38


def _pool_mat(n):
    i = np.arange(n)
    return ((np.abs(i[:, None] - i[None, :]) <= 1).astype(np.float32) / 3.0)


def _upsample_mat(n_in):
    eye = jnp.eye(n_in, dtype=jnp.float32)
    return jax.image.resize(eye, (H, n_in), method="bilinear")


def _desc_body(p1_ref, p2_ref, p3_ref, w_ref, b_ref, s2t_ref,
               s3t_ref, phi_ref):
    f32 = jnp.float32
    dn_ = (((1,), (0,)), ((), ()))
    lane = lax.broadcasted_iota(jnp.int32, (1, S), 1)
    wpos = lane % H
    q1 = lax.dot_general(w_ref[:, 0:C1], p1_ref[0], dn_,
                         preferred_element_type=f32)
    zc = jnp.zeros((D, 1), f32)
    lft = jnp.concatenate([q1[:, 1:], zc], axis=1)
    rgt = jnp.concatenate([zc, q1[:, :-1]], axis=1)
    lft = jnp.where(wpos == (H - 1), f32(0.0), lft)
    rgt = jnp.where(wpos == 0, f32(0.0), rgt)
    pw = (q1 + lft + rgt) * f32(1.0 / 3.0)
    zr = jnp.zeros((D, H), f32)
    up = jnp.concatenate([pw[:, H:], zr], axis=1)
    dn = jnp.concatenate([zr, pw[:, :-H]], axis=1)
    a1 = (pw + up + dn) * f32(1.0 / 3.0)
    q2 = lax.dot_general(w_ref[:, C1:C1 + C2], p2_ref[0], dn_,
                         preferred_element_type=f32)
    a2 = lax.dot_general(q2, s2t_ref[...], dn_, preferred_element_type=f32)
    q3 = lax.dot_general(w_ref[:, C1 + C2:C1 + C2 + C3], p3_ref[0], dn_,
                         preferred_element_type=f32)
    a3 = lax.dot_general(q3, s3t_ref[...], dn_, preferred_element_type=f32)
    sc = f32(2.0 / (H - 1))
    xx = wpos.astype(f32) * sc - 1.0
    yy = (lane // H).astype(f32) * sc - 1.0
    bias = (w_ref[:, 1792:1793] * xx + w_ref[:, 1793:1794] * yy
            + b_ref[...])
    phi_ref[0] = a1 + a2 + a3 + bias


def _knn_body(phi_ref, c_ref, out_ref):
    f32 = jnp.float32
    ph = phi_ref[0]
    cb = c_ref[...]
    feat = jnp.sum(ph * ph, axis=1, keepdims=True)
    cent = jnp.sum(cb * cb, axis=0, keepdims=True)
    cross = lax.dot_general(ph, cb, (((1,), (0,)), ((), ())),
                            preferred_element_type=f32)
    d2 = feat + cent - 2.0 * cross
    cols = lax.broadcasted_iota(jnp.int32, (R, S), 1)
    m1 = jnp.min(d2, axis=1, keepdims=True)
    i1 = jnp.min(jnp.where(d2 == m1, cols, S), axis=1, keepdims=True)
    d2 = jnp.where(cols == i1, BIG, d2)
    m2 = jnp.min(d2, axis=1, keepdims=True)
    i2 = jnp.min(jnp.where(d2 == m2, cols, S), axis=1, keepdims=True)
    d2 = jnp.where(cols == i2, BIG, d2)
    m3 = jnp.min(d2, axis=1, keepdims=True)
    d1 = jnp.sqrt(jnp.maximum(m1, 1e-12))
    d2s = jnp.sqrt(jnp.maximum(m2, 1e-12))
    d3s = jnp.sqrt(jnp.maximum(m3, 1e-12))
    out_ref[0] = d1 / (1.0 + jnp.exp(d1 - d2s) + jnp.exp(d1 - d3s))


def kernel(p1, p2, p3, W, b, C_bank):
    p1r = p1.reshape(B, C1, S)
    p2r = p2.reshape(B, C2, S2)
    p3r = p3.reshape(B, C3, S3)
    b2 = b.reshape(D, 1)
    a28 = jnp.asarray(_pool_mat(28))
    a14 = jnp.asarray(_pool_mat(14))
    m2 = _upsample_mat(28) @ a28
    m3 = _upsample_mat(14) @ a14
    s2t = jnp.kron(m2, m2).T
    s3t = jnp.kron(m3, m3).T

    phi = pl.pallas_call(
        _desc_body,
        grid=(B,),
        in_specs=[
            pl.BlockSpec((1, C1, S), lambda i: (i, 0, 0)),
            pl.BlockSpec((1, C2, S2), lambda i: (i, 0, 0)),
            pl.BlockSpec((1, C3, S3), lambda i: (i, 0, 0)),
            pl.BlockSpec((D, 1794), lambda i: (0, 0)),
            pl.BlockSpec((D, 1), lambda i: (0, 0)),
            pl.BlockSpec((S2, S), lambda i: (0, 0)),
            pl.BlockSpec((S3, S), lambda i: (0, 0)),
        ],
        out_specs=pl.BlockSpec((1, D, S), lambda i: (i, 0, 0)),
        out_shape=jax.ShapeDtypeStruct((B, D, S), jnp.float32),
    )(p1r, p2r, p3r, W, b2, s2t, s3t)

    phi_p = phi.transpose(0, 2, 1)

    score = pl.pallas_call(
        _knn_body,
        grid=(B, S // R),
        in_specs=[
            pl.BlockSpec((1, R, D), lambda i, j: (i, j, 0)),
            pl.BlockSpec((D, S), lambda i, j: (0, 0)),
        ],
        out_specs=pl.BlockSpec((1, R, 1), lambda i, j: (i, j, 0)),
        out_shape=jax.ShapeDtypeStruct((B, S, 1), jnp.float32),
    )(phi_p, C_bank)
    return score.reshape(B, 1, H, H)


# speedup vs baseline: 71.6345x; 1.2755x over previous
import numpy as np
import jax
import jax.numpy as jnp
from jax import lax
from jax.experimental import pallas as pl
from jax.experimental.pallas import tpu as pltpu

B = 8
H = 56
S = H * H
D = 224
C1, C2, C3 = 256, 512, 1024
S2, S3 = 28 * 28, 14 * 14
KNN = 3
R = 784
BIG = 3.0e38


def _pool_mat(n):
    i = np.arange(n)
    return ((np.abs(i[:, None] - i[None, :]) <= 1).astype(np.float32) / 3.0)


def _upsample_mat(n_in):
    eye = jnp.eye(n_in, dtype=jnp.float32)
    return jax.image.resize(eye, (H, n_in), method="bilinear")


def _desc_body(p1_ref, p2_ref, p3_ref, w_ref, b_ref, s2t_ref,
               s3t_ref, phi_ref):
    f32 = jnp.float32
    dn_ = (((1,), (0,)), ((), ()))
    lane = lax.broadcasted_iota(jnp.int32, (1, S), 1)
    wpos = lane % H
    q1 = lax.dot_general(w_ref[:, 0:C1], p1_ref[0], dn_,
                         preferred_element_type=f32)
    zc = jnp.zeros((D, 1), f32)
    lft = jnp.concatenate([q1[:, 1:], zc], axis=1)
    rgt = jnp.concatenate([zc, q1[:, :-1]], axis=1)
    lft = jnp.where(wpos == (H - 1), f32(0.0), lft)
    rgt = jnp.where(wpos == 0, f32(0.0), rgt)
    pw = (q1 + lft + rgt) * f32(1.0 / 3.0)
    zr = jnp.zeros((D, H), f32)
    up = jnp.concatenate([pw[:, H:], zr], axis=1)
    dn = jnp.concatenate([zr, pw[:, :-H]], axis=1)
    a1 = (pw + up + dn) * f32(1.0 / 3.0)
    q2 = lax.dot_general(w_ref[:, C1:C1 + C2], p2_ref[0], dn_,
                         preferred_element_type=f32)
    a2 = lax.dot_general(q2, s2t_ref[...], dn_, preferred_element_type=f32)
    q3 = lax.dot_general(w_ref[:, C1 + C2:C1 + C2 + C3], p3_ref[0], dn_,
                         preferred_element_type=f32)
    a3 = lax.dot_general(q3, s3t_ref[...], dn_, preferred_element_type=f32)
    sc = f32(2.0 / (H - 1))
    xx = wpos.astype(f32) * sc - 1.0
    yy = (lane // H).astype(f32) * sc - 1.0
    bias = (w_ref[:, 1792:1793] * xx + w_ref[:, 1793:1794] * yy
            + b_ref[...])
    phi_ref[0] = a1 + a2 + a3 + bias


def _knn_body(phi_ref, c_ref, out_ref, cent_ref):
    f32 = jnp.float32
    i = pl.program_id(0)
    j = pl.program_id(1)
    ph = phi_ref[0]
    cb = c_ref[...]

    @pl.when(jnp.logical_and(i == 0, j == 0))
    def _():
        cent_ref[...] = jnp.sum(cb * cb, axis=0, keepdims=True)

    cent = cent_ref[...]
    feat = jnp.sum(ph * ph, axis=1, keepdims=True)
    ph2 = ph * f32(-2.0)
    dn_ = (((1,), (0,)), ((), ()))

    a1 = jnp.full((R, 128), BIG, f32)
    a2 = a1
    a3 = a1
    CH = 512
    for lo in range(0, S, CH):
        hi = min(lo + CH, S)
        crossk = lax.dot_general(ph2, cb[:, lo:hi], dn_,
                                 preferred_element_type=f32)
        d2k = (crossk + cent[:, lo:hi]) + feat
        for m in range(0, hi - lo, 128):
            v = d2k[:, m:m + 128]
            if v.shape[1] < 128:
                v = jnp.concatenate(
                    [v, jnp.full((R, 128 - v.shape[1]), BIG, f32)], axis=1)
            t2 = jnp.maximum(a1, v)
            a1 = jnp.minimum(a1, v)
            t4 = jnp.maximum(a2, t2)
            a2 = jnp.minimum(a2, t2)
            a3 = jnp.minimum(a3, t4)
    c = jnp.concatenate([a1, a2, a3], axis=1)
    cols = lax.broadcasted_iota(jnp.int32, (R, 384), 1)
    m1 = jnp.min(c, axis=1, keepdims=True)
    i1 = jnp.min(jnp.where(c == m1, cols, 384), axis=1, keepdims=True)
    c = jnp.where(cols == i1, BIG, c)
    m2 = jnp.min(c, axis=1, keepdims=True)
    i2 = jnp.min(jnp.where(c == m2, cols, 384), axis=1, keepdims=True)
    c = jnp.where(cols == i2, BIG, c)
    m3 = jnp.min(c, axis=1, keepdims=True)
    d1 = jnp.sqrt(jnp.maximum(m1, 1e-12))
    d2s = jnp.sqrt(jnp.maximum(m2, 1e-12))
    d3s = jnp.sqrt(jnp.maximum(m3, 1e-12))
    out_ref[0] = d1 / (1.0 + jnp.exp(d1 - d2s) + jnp.exp(d1 - d3s))


def kernel(p1, p2, p3, W, b, C_bank):
    p1r = p1.reshape(B, C1, S)
    p2r = p2.reshape(B, C2, S2)
    p3r = p3.reshape(B, C3, S3)
    b2 = b.reshape(D, 1)
    a28 = jnp.asarray(_pool_mat(28))
    a14 = jnp.asarray(_pool_mat(14))
    m2 = _upsample_mat(28) @ a28
    m3 = _upsample_mat(14) @ a14
    s2t = jnp.kron(m2, m2).T
    s3t = jnp.kron(m3, m3).T

    phi = pl.pallas_call(
        _desc_body,
        grid=(B,),
        in_specs=[
            pl.BlockSpec((1, C1, S), lambda i: (i, 0, 0)),
            pl.BlockSpec((1, C2, S2), lambda i: (i, 0, 0)),
            pl.BlockSpec((1, C3, S3), lambda i: (i, 0, 0)),
            pl.BlockSpec((D, 1794), lambda i: (0, 0)),
            pl.BlockSpec((D, 1), lambda i: (0, 0)),
            pl.BlockSpec((S2, S), lambda i: (0, 0)),
            pl.BlockSpec((S3, S), lambda i: (0, 0)),
        ],
        out_specs=pl.BlockSpec((1, D, S), lambda i: (i, 0, 0)),
        out_shape=jax.ShapeDtypeStruct((B, D, S), jnp.float32),
    )(p1r, p2r, p3r, W, b2, s2t, s3t)

    phi_p = phi.transpose(0, 2, 1)

    score = pl.pallas_call(
        _knn_body,
        grid=(B, S // R),
        in_specs=[
            pl.BlockSpec((1, R, D), lambda i, j: (i, j, 0)),
            pl.BlockSpec((D, S), lambda i, j: (0, 0)),
        ],
        out_specs=pl.BlockSpec((1, R, 1), lambda i, j: (i, j, 0)),
        out_shape=jax.ShapeDtypeStruct((B, S, 1), jnp.float32),
        scratch_shapes=[pltpu.VMEM((1, S), jnp.float32)],
    )(phi_p, C_bank)
    return score.reshape(B, 1, H, H)


# speedup vs baseline: 78.2823x; 1.0928x over previous
import numpy as np
import jax
import jax.numpy as jnp
from jax import lax
from jax.experimental import pallas as pl
from jax.experimental.pallas import tpu as pltpu

B = 8
H = 56
S = H * H
D = 224
C1, C2, C3 = 256, 512, 1024
S2, S3 = 28 * 28, 14 * 14
KNN = 3
R = 784
BIG = 3.0e38


def _pool_mat(n):
    i = np.arange(n)
    return ((np.abs(i[:, None] - i[None, :]) <= 1).astype(np.float32) / 3.0)


def _upsample_mat(n_in):
    eye = jnp.eye(n_in, dtype=jnp.float32)
    return jax.image.resize(eye, (H, n_in), method="bilinear")


def _desc_body(p1_ref, p2_ref, p3_ref, w_ref, b_ref, s2t_ref,
               s3t_ref, cb_ref, phi_ref, cent_ref):
    f32 = jnp.float32
    dn_ = (((1,), (0,)), ((), ()))
    lane = lax.broadcasted_iota(jnp.int32, (1, S), 1)
    wpos = lane % H
    q1 = lax.dot_general(w_ref[:, 0:C1], p1_ref[0], dn_,
                         preferred_element_type=f32)
    zc = jnp.zeros((D, 1), f32)
    lft = jnp.concatenate([q1[:, 1:], zc], axis=1)
    rgt = jnp.concatenate([zc, q1[:, :-1]], axis=1)
    lft = jnp.where(wpos == (H - 1), f32(0.0), lft)
    rgt = jnp.where(wpos == 0, f32(0.0), rgt)
    pw = (q1 + lft + rgt) * f32(1.0 / 3.0)
    zr = jnp.zeros((D, H), f32)
    up = jnp.concatenate([pw[:, H:], zr], axis=1)
    dn = jnp.concatenate([zr, pw[:, :-H]], axis=1)
    a1 = (pw + up + dn) * f32(1.0 / 3.0)
    q2 = lax.dot_general(w_ref[:, C1:C1 + C2], p2_ref[0], dn_,
                         preferred_element_type=f32)
    a2 = lax.dot_general(q2, s2t_ref[...], dn_, preferred_element_type=f32)
    q3 = lax.dot_general(w_ref[:, C1 + C2:C1 + C2 + C3], p3_ref[0], dn_,
                         preferred_element_type=f32)
    a3 = lax.dot_general(q3, s3t_ref[...], dn_, preferred_element_type=f32)
    sc = f32(2.0 / (H - 1))
    xx = wpos.astype(f32) * sc - 1.0
    yy = (lane // H).astype(f32) * sc - 1.0
    bias = (w_ref[:, 1792:1793] * xx + w_ref[:, 1793:1794] * yy
            + b_ref[...])
    phi = (a1 + a2 + a3 + bias) * f32(-2.0)
    phi_ref[0] = jnp.concatenate([phi, jnp.ones((1, S), f32)], axis=0)

    @pl.when(pl.program_id(0) == 0)
    def _():
        cb = cb_ref[...]
        cent_ref[...] = jnp.sum(cb * cb, axis=0, keepdims=True)


def _knn_body(phi_ref, c_ref, out_ref):
    f32 = jnp.float32
    ph2 = phi_ref[0]
    cba = c_ref[...]
    phq = ph2[:, 0:D]
    feat = jnp.sum(phq * phq, axis=1, keepdims=True) * f32(0.25)
    dn_ = (((1,), (0,)), ((), ()))

    a1 = jnp.full((R, 128), BIG, f32)
    a2 = a1
    a3 = a1
    CH = 512
    for lo in range(0, S, CH):
        hi = min(lo + CH, S)
        d2k = lax.dot_general(ph2, cba[:, lo:hi], dn_,
                              preferred_element_type=f32)
        for m in range(0, hi - lo, 128):
            v = d2k[:, m:m + 128]
            if v.shape[1] < 128:
                v = jnp.concatenate(
                    [v, jnp.full((R, 128 - v.shape[1]), BIG, f32)], axis=1)
            t2 = jnp.maximum(a1, v)
            a1 = jnp.minimum(a1, v)
            t4 = jnp.maximum(a2, t2)
            a2 = jnp.minimum(a2, t2)
            a3 = jnp.minimum(a3, t4)
    c = jnp.concatenate([a1, a2, a3], axis=1)
    cols = lax.broadcasted_iota(jnp.int32, (R, 384), 1)
    m1 = jnp.min(c, axis=1, keepdims=True)
    i1 = jnp.min(jnp.where(c == m1, cols, 384), axis=1, keepdims=True)
    c = jnp.where(cols == i1, BIG, c)
    m2 = jnp.min(c, axis=1, keepdims=True)
    i2 = jnp.min(jnp.where(c == m2, cols, 384), axis=1, keepdims=True)
    c = jnp.where(cols == i2, BIG, c)
    m3 = jnp.min(c, axis=1, keepdims=True)
    d1 = jnp.sqrt(jnp.maximum(m1 + feat, 1e-12))
    d2s = jnp.sqrt(jnp.maximum(m2 + feat, 1e-12))
    d3s = jnp.sqrt(jnp.maximum(m3 + feat, 1e-12))
    out_ref[0] = d1 / (1.0 + jnp.exp(d1 - d2s) + jnp.exp(d1 - d3s))


def kernel(p1, p2, p3, W, b, C_bank):
    p1r = p1.reshape(B, C1, S)
    p2r = p2.reshape(B, C2, S2)
    p3r = p3.reshape(B, C3, S3)
    b2 = b.reshape(D, 1)
    a28 = jnp.asarray(_pool_mat(28))
    a14 = jnp.asarray(_pool_mat(14))
    m2 = _upsample_mat(28) @ a28
    m3 = _upsample_mat(14) @ a14
    s2t = jnp.kron(m2, m2).T
    s3t = jnp.kron(m3, m3).T

    phi, cent = pl.pallas_call(
        _desc_body,
        grid=(B,),
        in_specs=[
            pl.BlockSpec((1, C1, S), lambda i: (i, 0, 0)),
            pl.BlockSpec((1, C2, S2), lambda i: (i, 0, 0)),
            pl.BlockSpec((1, C3, S3), lambda i: (i, 0, 0)),
            pl.BlockSpec((D, 1794), lambda i: (0, 0)),
            pl.BlockSpec((D, 1), lambda i: (0, 0)),
            pl.BlockSpec((S2, S), lambda i: (0, 0)),
            pl.BlockSpec((S3, S), lambda i: (0, 0)),
            pl.BlockSpec((D, S), lambda i: (0, 0)),
        ],
        out_specs=[
            pl.BlockSpec((1, D + 1, S), lambda i: (i, 0, 0)),
            pl.BlockSpec((1, S), lambda i: (0, 0)),
        ],
        out_shape=[
            jax.ShapeDtypeStruct((B, D + 1, S), jnp.float32),
            jax.ShapeDtypeStruct((1, S), jnp.float32),
        ],
    )(p1r, p2r, p3r, W, b2, s2t, s3t, C_bank)

    phi_p = phi.transpose(0, 2, 1)
    cb_aug = jnp.concatenate([C_bank, cent], axis=0)

    score = pl.pallas_call(
        _knn_body,
        grid=(B, S // R),
        in_specs=[
            pl.BlockSpec((1, R, D + 1), lambda i, j: (i, j, 0)),
            pl.BlockSpec((D + 1, S), lambda i, j: (0, 0)),
        ],
        out_specs=pl.BlockSpec((1, R, 1), lambda i, j: (i, j, 0)),
        out_shape=jax.ShapeDtypeStruct((B, S, 1), jnp.float32),
    )(phi_p, cb_aug)
    return score.reshape(B, 1, H, H)


# speedup vs baseline: 116.2848x; 1.4855x over previous
import numpy as np
import jax
import jax.numpy as jnp
from jax import lax
from jax.experimental import pallas as pl

B = 8
H = 56
S = H * H
D = 224
C1, C2, C3 = 256, 512, 1024
S2, S3 = 28 * 28, 14 * 14
R = 784
BIG = 3.0e38
NC = S2 + S3 + 3


def _pool_mat(n):
    i = np.arange(n)
    return (np.abs(i[:, None] - i[None, :]) <= 1).astype(np.float64) / 3.0


def _up_mat(n_in):
    scale = H / n_in
    x = (np.arange(H) + 0.5) / scale - 0.5
    w = np.maximum(0.0, 1.0 - np.abs(x[:, None] - np.arange(n_in)[None, :]))
    return w / w.sum(axis=1, keepdims=True)


def _build_rhs():
    m2 = _up_mat(28) @ _pool_mat(28)
    m3 = _up_mat(14) @ _pool_mat(14)
    s2t = np.kron(m2, m2).T
    s3t = np.kron(m3, m3).T
    ss = np.arange(S)
    xx = (-1.0 + 2.0 * (ss % H) / (H - 1))[None, :]
    yy = (-1.0 + 2.0 * (ss // H) / (H - 1))[None, :]
    ones = np.ones((1, S))
    rhs = np.concatenate([s2t, s3t, xx, yy, ones], axis=0)
    return (-2.0 * rhs).astype(np.float32)


_RHS = _build_rhs()


def _desc_body(p1_ref, p2_ref, p3_ref, w_ref, b_ref, rhs_ref, cbt_ref,
               phi_ref, cent_ref):
    f32 = jnp.float32
    dn_ = (((1,), (0,)), ((), ()))
    lane = lax.broadcasted_iota(jnp.int32, (1, S), 1)
    wpos = lane % H
    w1s = w_ref[:, 0:C1] * f32(-2.0)
    q1 = lax.dot_general(w1s, p1_ref[0], dn_,
                         preferred_element_type=f32)
    zc = jnp.zeros((D, 1), f32)
    lft = jnp.concatenate([q1[:, 1:], zc], axis=1)
    rgt = jnp.concatenate([zc, q1[:, :-1]], axis=1)
    lft = jnp.where(wpos == (H - 1), f32(0.0), lft)
    rgt = jnp.where(wpos == 0, f32(0.0), rgt)
    pw = (q1 + lft + rgt) * f32(1.0 / 3.0)
    zr = jnp.zeros((D, H), f32)
    up = jnp.concatenate([pw[:, H:], zr], axis=1)
    dn = jnp.concatenate([zr, pw[:, :-H]], axis=1)
    a1 = (pw + up + dn) * f32(1.0 / 3.0)
    q2 = lax.dot_general(w_ref[:, C1:C1 + C2], p2_ref[0], dn_,
                         preferred_element_type=f32)
    q3 = lax.dot_general(w_ref[:, C1 + C2:C1 + C2 + C3], p3_ref[0], dn_,
                         preferred_element_type=f32)
    lhs = jnp.concatenate(
        [q2, q3, w_ref[:, 1792:1793], w_ref[:, 1793:1794], b_ref[...]],
        axis=1)
    rest = lax.dot_general(lhs, rhs_ref[...], dn_,
                           preferred_element_type=f32)
    phi_ref[0] = a1 + rest

    @pl.when(pl.program_id(0) == 0)
    def _():
        cbt = cbt_ref[...]
        cent_ref[...] = jnp.sum(cbt * cbt, axis=1, keepdims=True)


def _knn_body(phi_ref, c_ref, cent_ref, out_ref):
    f32 = jnp.float32
    ph2 = phi_ref[0]
    cb = c_ref[...]
    cent = cent_ref[...]
    feat = jnp.sum(ph2 * ph2, axis=0, keepdims=True) * f32(0.25)
    dn_ = (((0,), (0,)), ((), ()))

    a1 = jnp.full((32, S), BIG, f32)
    a2 = a1
    a3 = a1
    CH = 512
    for lo in range(0, S, CH):
        hi = min(lo + CH, S)
        d2k = lax.dot_general(cb[:, lo:hi], ph2, dn_,
                              preferred_element_type=f32) + cent[lo:hi]
        for m in range(0, hi - lo, 32):
            v = d2k[m:m + 32, :]
            t2 = jnp.maximum(a1, v)
            a1 = jnp.minimum(a1, v)
            t4 = jnp.maximum(a2, t2)
            a2 = jnp.minimum(a2, t2)
            a3 = jnp.minimum(a3, t4)
    c = jnp.concatenate([a1, a2, a3], axis=0)
    rows = lax.broadcasted_iota(jnp.int32, (96, S), 0)
    m1 = jnp.min(c, axis=0, keepdims=True)
    i1 = jnp.min(jnp.where(c == m1, rows, 96), axis=0, keepdims=True)
    c = jnp.where(rows == i1, BIG, c)
    m2 = jnp.min(c, axis=0, keepdims=True)
    i2 = jnp.min(jnp.where(c == m2, rows, 96), axis=0, keepdims=True)
    c = jnp.where(rows == i2, BIG, c)
    m3 = jnp.min(c, axis=0, keepdims=True)
    d1 = jnp.sqrt(jnp.maximum(m1 + feat, 1e-12))
    d2s = jnp.sqrt(jnp.maximum(m2 + feat, 1e-12))
    d3s = jnp.sqrt(jnp.maximum(m3 + feat, 1e-12))
    out_ref[0] = d1 / (1.0 + jnp.exp(d1 - d2s) + jnp.exp(d1 - d3s))


def kernel(p1, p2, p3, W, b, C_bank):
    p1r = p1.reshape(B, C1, S)
    p2r = p2.reshape(B, C2, S2)
    p3r = p3.reshape(B, C3, S3)
    b2 = b.reshape(D, 1)
    rhs = jnp.asarray(_RHS)
    cbt = C_bank.T

    phi, cent = pl.pallas_call(
        _desc_body,
        grid=(B,),
        in_specs=[
            pl.BlockSpec((1, C1, S), lambda i: (i, 0, 0)),
            pl.BlockSpec((1, C2, S2), lambda i: (i, 0, 0)),
            pl.BlockSpec((1, C3, S3), lambda i: (i, 0, 0)),
            pl.BlockSpec((D, 1794), lambda i: (0, 0)),
            pl.BlockSpec((D, 1), lambda i: (0, 0)),
            pl.BlockSpec((NC, S), lambda i: (0, 0)),
            pl.BlockSpec((S, D), lambda i: (0, 0)),
        ],
        out_specs=[
            pl.BlockSpec((1, D, S), lambda i: (i, 0, 0)),
            pl.BlockSpec((S, 1), lambda i: (0, 0)),
        ],
        out_shape=[
            jax.ShapeDtypeStruct((B, D, S), jnp.float32),
            jax.ShapeDtypeStruct((S, 1), jnp.float32),
        ],
    )(p1r, p2r, p3r, W, b2, rhs, cbt)

    score = pl.pallas_call(
        _knn_body,
        grid=(B,),
        in_specs=[
            pl.BlockSpec((1, D, S), lambda i: (i, 0, 0)),
            pl.BlockSpec((D, S), lambda i: (0, 0)),
            pl.BlockSpec((S, 1), lambda i: (0, 0)),
        ],
        out_specs=pl.BlockSpec((1, 1, S), lambda i: (i, 0, 0)),
        out_shape=jax.ShapeDtypeStruct((B, 1, S), jnp.float32),
    )(phi, C_bank, cent)
    return score.reshape(B, 1, H, H)


# speedup vs baseline: 119.7362x; 1.0297x over previous
import numpy as np
import jax
import jax.numpy as jnp
from jax import lax
from jax.experimental import pallas as pl

B = 8
H = 56
S = H * H
D = 224
C1, C2, C3 = 256, 512, 1024
S2, S3 = 28 * 28, 14 * 14
BIG = 3.0e38
NC = S2 + S3 + 3


def _pool_mat(n):
    i = np.arange(n)
    return (np.abs(i[:, None] - i[None, :]) <= 1).astype(np.float64) / 3.0


def _up_mat(n_in):
    scale = H / n_in
    x = (np.arange(H) + 0.5) / scale - 0.5
    w = np.maximum(0.0, 1.0 - np.abs(x[:, None] - np.arange(n_in)[None, :]))
    return w / w.sum(axis=1, keepdims=True)


def _build_rhs():
    m2 = _up_mat(28) @ _pool_mat(28)
    m3 = _up_mat(14) @ _pool_mat(14)
    s2t = np.kron(m2, m2).T
    s3t = np.kron(m3, m3).T
    ss = np.arange(S)
    xx = (-1.0 + 2.0 * (ss % H) / (H - 1))[None, :]
    yy = (-1.0 + 2.0 * (ss // H) / (H - 1))[None, :]
    ones = np.ones((1, S))
    rhs = np.concatenate([s2t, s3t, xx, yy, ones], axis=0)
    return (-2.0 * rhs).astype(np.float32)


_RHS = _build_rhs()


def _fused_body(p1_ref, p2_ref, p3_ref, w_ref, b_ref, rhs_ref, cb_ref,
                cbt_ref, out_ref):
    f32 = jnp.float32
    dn_ = (((1,), (0,)), ((), ()))
    lane = lax.broadcasted_iota(jnp.int32, (1, S), 1)
    wpos = lane % H
    w1s = w_ref[:, 0:C1] * f32(-2.0)
    q1 = lax.dot_general(w1s, p1_ref[0], dn_,
                         preferred_element_type=f32)
    zc = jnp.zeros((D, 1), f32)
    lft = jnp.concatenate([q1[:, 1:], zc], axis=1)
    rgt = jnp.concatenate([zc, q1[:, :-1]], axis=1)
    lft = jnp.where(wpos == (H - 1), f32(0.0), lft)
    rgt = jnp.where(wpos == 0, f32(0.0), rgt)
    pw = (q1 + lft + rgt) * f32(1.0 / 3.0)
    zr = jnp.zeros((D, H), f32)
    up = jnp.concatenate([pw[:, H:], zr], axis=1)
    dn = jnp.concatenate([zr, pw[:, :-H]], axis=1)
    a1 = (pw + up + dn) * f32(1.0 / 3.0)
    q2 = lax.dot_general(w_ref[:, C1:C1 + C2], p2_ref[0], dn_,
                         preferred_element_type=f32)
    q3 = lax.dot_general(w_ref[:, C1 + C2:C1 + C2 + C3], p3_ref[0], dn_,
                         preferred_element_type=f32)
    lhs = jnp.concatenate(
        [q2, q3, w_ref[:, 1792:1793], w_ref[:, 1793:1794], b_ref[...]],
        axis=1)
    rest = lax.dot_general(lhs, rhs_ref[...], dn_,
                           preferred_element_type=f32)
    ph2 = a1 + rest

    cb = cb_ref[...]
    cbt = cbt_ref[...]
    cent = jnp.sum(cbt * cbt, axis=1, keepdims=True)
    feat = jnp.sum(ph2 * ph2, axis=0, keepdims=True) * f32(0.25)
    dnt = (((0,), (0,)), ((), ()))

    a1_ = jnp.full((32, S), BIG, f32)
    a2_ = a1_
    a3_ = a1_
    CH = 512
    for lo in range(0, S, CH):
        hi = min(lo + CH, S)
        d2k = lax.dot_general(cb[:, lo:hi], ph2, dnt,
                              preferred_element_type=f32) + cent[lo:hi]
        for m in range(0, hi - lo, 32):
            v = d2k[m:m + 32, :]
            t2 = jnp.maximum(a1_, v)
            a1_ = jnp.minimum(a1_, v)
            t4 = jnp.maximum(a2_, t2)
            a2_ = jnp.minimum(a2_, t2)
            a3_ = jnp.minimum(a3_, t4)
    c = jnp.concatenate([a1_, a2_, a3_], axis=0)
    rows = lax.broadcasted_iota(jnp.int32, (96, S), 0)
    m1 = jnp.min(c, axis=0, keepdims=True)
    i1 = jnp.min(jnp.where(c == m1, rows, 96), axis=0, keepdims=True)
    c = jnp.where(rows == i1, BIG, c)
    m2 = jnp.min(c, axis=0, keepdims=True)
    i2 = jnp.min(jnp.where(c == m2, rows, 96), axis=0, keepdims=True)
    c = jnp.where(rows == i2, BIG, c)
    m3 = jnp.min(c, axis=0, keepdims=True)
    d1 = jnp.sqrt(jnp.maximum(m1 + feat, 1e-12))
    d2s = jnp.sqrt(jnp.maximum(m2 + feat, 1e-12))
    d3s = jnp.sqrt(jnp.maximum(m3 + feat, 1e-12))
    out_ref[0] = d1 / (1.0 + jnp.exp(d1 - d2s) + jnp.exp(d1 - d3s))


def kernel(p1, p2, p3, W, b, C_bank):
    p1r = p1.reshape(B, C1, S)
    p2r = p2.reshape(B, C2, S2)
    p3r = p3.reshape(B, C3, S3)
    b2 = b.reshape(D, 1)
    rhs = jnp.asarray(_RHS)
    cbt = C_bank.T

    score = pl.pallas_call(
        _fused_body,
        grid=(B,),
        in_specs=[
            pl.BlockSpec((1, C1, S), lambda i: (i, 0, 0)),
            pl.BlockSpec((1, C2, S2), lambda i: (i, 0, 0)),
            pl.BlockSpec((1, C3, S3), lambda i: (i, 0, 0)),
            pl.BlockSpec((D, 1794), lambda i: (0, 0)),
            pl.BlockSpec((D, 1), lambda i: (0, 0)),
            pl.BlockSpec((NC, S), lambda i: (0, 0)),
            pl.BlockSpec((D, S), lambda i: (0, 0)),
            pl.BlockSpec((S, D), lambda i: (0, 0)),
        ],
        out_specs=pl.BlockSpec((1, 1, S), lambda i: (i, 0, 0)),
        out_shape=jax.ShapeDtypeStruct((B, 1, S), jnp.float32),
    )(p1r, p2r, p3r, W, b2, rhs, C_bank, cbt)
    return score.reshape(B, 1, H, H)


# speedup vs baseline: 120.3111x; 1.0048x over previous
import numpy as np
import jax
import jax.numpy as jnp
from jax import lax
from jax.experimental import pallas as pl
from jax.experimental.pallas import tpu as pltpu

B = 8
H = 56
S = H * H
D = 224
C1, C2, C3 = 256, 512, 1024
S2, S3 = 28 * 28, 14 * 14
BIG = 3.0e38
NC = S2 + S3 + 3


def _pool_mat(n):
    i = np.arange(n)
    return (np.abs(i[:, None] - i[None, :]) <= 1).astype(np.float64) / 3.0


def _up_mat(n_in):
    scale = H / n_in
    x = (np.arange(H) + 0.5) / scale - 0.5
    w = np.maximum(0.0, 1.0 - np.abs(x[:, None] - np.arange(n_in)[None, :]))
    return w / w.sum(axis=1, keepdims=True)


def _build_rhs():
    m2 = _up_mat(28) @ _pool_mat(28)
    m3 = _up_mat(14) @ _pool_mat(14)
    s2t = np.kron(m2, m2).T
    s3t = np.kron(m3, m3).T
    ss = np.arange(S)
    xx = (-1.0 + 2.0 * (ss % H) / (H - 1))[None, :]
    yy = (-1.0 + 2.0 * (ss // H) / (H - 1))[None, :]
    ones = np.ones((1, S))
    rhs = np.concatenate([s2t, s3t, xx, yy, ones], axis=0)
    return (-2.0 * rhs).astype(np.float32)


_RHS = _build_rhs()


def _fused_body(p1_ref, p2_ref, p3_ref, w_ref, b_ref, rhs_ref, cb_ref,
                cbt_ref, out_ref):
    f32 = jnp.float32
    dn_ = (((1,), (0,)), ((), ()))
    lane = lax.broadcasted_iota(jnp.int32, (1, S), 1)
    wpos = lane % H
    w1s = w_ref[:, 0:C1] * f32(-2.0)
    q1 = lax.dot_general(w1s, p1_ref[0], dn_,
                         preferred_element_type=f32)
    zc = jnp.zeros((D, 1), f32)
    lft = jnp.concatenate([q1[:, 1:], zc], axis=1)
    rgt = jnp.concatenate([zc, q1[:, :-1]], axis=1)
    lft = jnp.where(wpos == (H - 1), f32(0.0), lft)
    rgt = jnp.where(wpos == 0, f32(0.0), rgt)
    pw = (q1 + lft + rgt) * f32(1.0 / 3.0)
    zr = jnp.zeros((D, H), f32)
    up = jnp.concatenate([pw[:, H:], zr], axis=1)
    dn = jnp.concatenate([zr, pw[:, :-H]], axis=1)
    a1 = (pw + up + dn) * f32(1.0 / 3.0)
    q2 = lax.dot_general(w_ref[:, C1:C1 + C2], p2_ref[0], dn_,
                         preferred_element_type=f32)
    q3 = lax.dot_general(w_ref[:, C1 + C2:C1 + C2 + C3], p3_ref[0], dn_,
                         preferred_element_type=f32)
    lhs = jnp.concatenate(
        [q2, q3, w_ref[:, 1792:1793], w_ref[:, 1793:1794], b_ref[...]],
        axis=1)
    rest = lax.dot_general(lhs, rhs_ref[...], dn_,
                           preferred_element_type=f32)
    ph2 = a1 + rest

    cb = cb_ref[...]
    cbt = cbt_ref[...]
    cent = jnp.sum(cbt * cbt, axis=1, keepdims=True)
    feat = jnp.sum(ph2 * ph2, axis=0, keepdims=True) * f32(0.25)
    dnt = (((0,), (0,)), ((), ()))

    a1_ = jnp.full((32, S), BIG, f32)
    a2_ = a1_
    a3_ = a1_
    CH = 512
    for lo in range(0, S, CH):
        hi = min(lo + CH, S)
        d2k = lax.dot_general(cb[:, lo:hi], ph2, dnt,
                              preferred_element_type=f32) + cent[lo:hi]
        for m in range(0, hi - lo, 32):
            v = d2k[m:m + 32, :]
            t2 = jnp.maximum(a1_, v)
            a1_ = jnp.minimum(a1_, v)
            t4 = jnp.maximum(a2_, t2)
            a2_ = jnp.minimum(a2_, t2)
            a3_ = jnp.minimum(a3_, t4)
    c = jnp.concatenate([a1_, a2_, a3_], axis=0)
    rows = lax.broadcasted_iota(jnp.int32, (96, S), 0)
    m1 = jnp.min(c, axis=0, keepdims=True)
    i1 = jnp.min(jnp.where(c == m1, rows, 96), axis=0, keepdims=True)
    c = jnp.where(rows == i1, BIG, c)
    m2 = jnp.min(c, axis=0, keepdims=True)
    i2 = jnp.min(jnp.where(c == m2, rows, 96), axis=0, keepdims=True)
    c = jnp.where(rows == i2, BIG, c)
    m3 = jnp.min(c, axis=0, keepdims=True)
    d1 = jnp.sqrt(jnp.maximum(m1 + feat, 1e-12))
    d2s = jnp.sqrt(jnp.maximum(m2 + feat, 1e-12))
    d3s = jnp.sqrt(jnp.maximum(m3 + feat, 1e-12))
    out_ref[0] = d1 / (1.0 + jnp.exp(d1 - d2s) + jnp.exp(d1 - d3s))


def kernel(p1, p2, p3, W, b, C_bank):
    p1r = p1.reshape(B, C1, S)
    p2r = p2.reshape(B, C2, S2)
    p3r = p3.reshape(B, C3, S3)
    b2 = b.reshape(D, 1)
    rhs = jnp.asarray(_RHS)
    cbt = C_bank.T

    score = pl.pallas_call(
        _fused_body,
        grid=(B,),
        in_specs=[
            pl.BlockSpec((1, C1, S), lambda i: (i, 0, 0)),
            pl.BlockSpec((1, C2, S2), lambda i: (i, 0, 0)),
            pl.BlockSpec((1, C3, S3), lambda i: (i, 0, 0)),
            pl.BlockSpec((D, 1794), lambda i: (0, 0)),
            pl.BlockSpec((D, 1), lambda i: (0, 0)),
            pl.BlockSpec((NC, S), lambda i: (0, 0)),
            pl.BlockSpec((D, S), lambda i: (0, 0)),
            pl.BlockSpec((S, D), lambda i: (0, 0)),
        ],
        out_specs=pl.BlockSpec((1, 1, S), lambda i: (i, 0, 0)),
        out_shape=jax.ShapeDtypeStruct((B, 1, S), jnp.float32),
        compiler_params=pltpu.CompilerParams(
            vmem_limit_bytes=128 * 1024 * 1024),
    )(p1r, p2r, p3r, W, b2, rhs, C_bank, cbt)
    return score.reshape(B, 1, H, H)


# speedup vs baseline: 122.9583x; 1.0220x over previous
import numpy as np
import jax
import jax.numpy as jnp
from jax import lax
from jax.experimental import pallas as pl
from jax.experimental.pallas import tpu as pltpu

B = 8
H = 56
S = H * H
D = 224
C1, C2, C3 = 256, 512, 1024
S2, S3 = 28 * 28, 14 * 14
BIG = 3.0e38
NC = S2 + S3 + 3


def _pool_mat(n):
    i = np.arange(n)
    return (np.abs(i[:, None] - i[None, :]) <= 1).astype(np.float64) / 3.0


def _up_mat(n_in):
    scale = H / n_in
    x = (np.arange(H) + 0.5) / scale - 0.5
    w = np.maximum(0.0, 1.0 - np.abs(x[:, None] - np.arange(n_in)[None, :]))
    return w / w.sum(axis=1, keepdims=True)


def _build_rhs():
    m2 = _up_mat(28) @ _pool_mat(28)
    m3 = _up_mat(14) @ _pool_mat(14)
    s2t = np.kron(m2, m2).T
    s3t = np.kron(m3, m3).T
    ss = np.arange(S)
    xx = (-1.0 + 2.0 * (ss % H) / (H - 1))[None, :]
    yy = (-1.0 + 2.0 * (ss // H) / (H - 1))[None, :]
    ones = np.ones((1, S))
    rhs = np.concatenate([s2t, s3t, xx, yy, ones], axis=0)
    return (-2.0 * rhs).astype(np.float32)


_RHS = _build_rhs()


def _fused_body(p1_ref, p2_ref, p3_ref, w_ref, b_ref, rhs_ref, cb_ref,
                cbt_ref, out_ref):
    f32 = jnp.float32
    dn_ = (((1,), (0,)), ((), ()))
    lane = lax.broadcasted_iota(jnp.int32, (1, S), 1)
    wpos = lane % H
    w1s = w_ref[:, 0:C1] * f32(-2.0)
    q1 = lax.dot_general(w1s, p1_ref[0], dn_,
                         preferred_element_type=f32)
    zc = jnp.zeros((D, 1), f32)
    lft = jnp.concatenate([q1[:, 1:], zc], axis=1)
    rgt = jnp.concatenate([zc, q1[:, :-1]], axis=1)
    lft = jnp.where(wpos == (H - 1), f32(0.0), lft)
    rgt = jnp.where(wpos == 0, f32(0.0), rgt)
    pw = (q1 + lft + rgt) * f32(1.0 / 3.0)
    zr = jnp.zeros((D, H), f32)
    up = jnp.concatenate([pw[:, H:], zr], axis=1)
    dn = jnp.concatenate([zr, pw[:, :-H]], axis=1)
    a1 = (pw + up + dn) * f32(1.0 / 3.0)
    q2 = lax.dot_general(w_ref[:, C1:C1 + C2], p2_ref[0], dn_,
                         preferred_element_type=f32)
    q3 = lax.dot_general(w_ref[:, C1 + C2:C1 + C2 + C3], p3_ref[0], dn_,
                         preferred_element_type=f32)
    lhs = jnp.concatenate(
        [q2, q3, w_ref[:, 1792:1793], w_ref[:, 1793:1794], b_ref[...]],
        axis=1)
    rest = lax.dot_general(lhs, rhs_ref[...], dn_,
                           preferred_element_type=f32)
    ph2 = a1 + rest

    cb = cb_ref[...]
    cbt = cbt_ref[...]
    cent = jnp.sum(cbt * cbt, axis=1, keepdims=True)
    feat = jnp.sum(ph2 * ph2, axis=0, keepdims=True) * f32(0.25)
    dnt = (((0,), (0,)), ((), ()))

    a1_ = jnp.full((32, S), BIG, f32)
    a2_ = a1_
    a3_ = a1_
    cbh = cb.astype(jnp.bfloat16)
    phh = ph2.astype(jnp.bfloat16)
    CH = 512
    for lo in range(0, S, CH):
        hi = min(lo + CH, S)
        d2k = lax.dot_general(cbh[:, lo:hi], phh, dnt,
                              preferred_element_type=f32) + cent[lo:hi]
        for m in range(0, hi - lo, 32):
            v = d2k[m:m + 32, :]
            t2 = jnp.maximum(a1_, v)
            a1_ = jnp.minimum(a1_, v)
            t4 = jnp.maximum(a2_, t2)
            a2_ = jnp.minimum(a2_, t2)
            a3_ = jnp.minimum(a3_, t4)
    c = jnp.concatenate([a1_, a2_, a3_], axis=0)
    rows = lax.broadcasted_iota(jnp.int32, (96, S), 0)
    m1 = jnp.min(c, axis=0, keepdims=True)
    i1 = jnp.min(jnp.where(c == m1, rows, 96), axis=0, keepdims=True)
    c = jnp.where(rows == i1, BIG, c)
    m2 = jnp.min(c, axis=0, keepdims=True)
    i2 = jnp.min(jnp.where(c == m2, rows, 96), axis=0, keepdims=True)
    c = jnp.where(rows == i2, BIG, c)
    m3 = jnp.min(c, axis=0, keepdims=True)
    d1 = jnp.sqrt(jnp.maximum(m1 + feat, 1e-12))
    d2s = jnp.sqrt(jnp.maximum(m2 + feat, 1e-12))
    d3s = jnp.sqrt(jnp.maximum(m3 + feat, 1e-12))
    out_ref[0] = d1 / (1.0 + jnp.exp(d1 - d2s) + jnp.exp(d1 - d3s))


def kernel(p1, p2, p3, W, b, C_bank):
    p1r = p1.reshape(B, C1, S)
    p2r = p2.reshape(B, C2, S2)
    p3r = p3.reshape(B, C3, S3)
    b2 = b.reshape(D, 1)
    rhs = jnp.asarray(_RHS)
    cbt = C_bank.T

    score = pl.pallas_call(
        _fused_body,
        grid=(B,),
        in_specs=[
            pl.BlockSpec((1, C1, S), lambda i: (i, 0, 0)),
            pl.BlockSpec((1, C2, S2), lambda i: (i, 0, 0)),
            pl.BlockSpec((1, C3, S3), lambda i: (i, 0, 0)),
            pl.BlockSpec((D, 1794), lambda i: (0, 0)),
            pl.BlockSpec((D, 1), lambda i: (0, 0)),
            pl.BlockSpec((NC, S), lambda i: (0, 0)),
            pl.BlockSpec((D, S), lambda i: (0, 0)),
            pl.BlockSpec((S, D), lambda i: (0, 0)),
        ],
        out_specs=pl.BlockSpec((1, 1, S), lambda i: (i, 0, 0)),
        out_shape=jax.ShapeDtypeStruct((B, 1, S), jnp.float32),
        compiler_params=pltpu.CompilerParams(
            vmem_limit_bytes=128 * 1024 * 1024),
    )(p1r, p2r, p3r, W, b2, rhs, C_bank, cbt)
    return score.reshape(B, 1, H, H)
